# baseline (speedup 1.0000x reference)
import functools

import jax
import jax.numpy as jnp
from jax import lax
from jax.experimental import pallas as pl
from jax.experimental.pallas import tpu as pltpu
from jax.experimental.pallas import tpu_sc as plsc

N_NODES = 10000
N_EDGES = 320000
D = 128
NPAD = 10240
NC = 2
NS = 16
NW = NC * NS
CHUNK = 128
CPT = ((N_EDGES + NW * CHUNK - 1) // (NW * CHUNK) + 1) // 2 * 2
EPAD = NW * CPT * CHUNK
ROWS_PER_TILE = NPAD // NS

_mesh = plsc.VectorSubcoreMesh(core_axis_name="c", subcore_axis_name="s")


@functools.partial(
    pl.kernel,
    out_type=jax.ShapeDtypeStruct((NC, NPAD, D), jnp.float32),
    mesh=_mesh,
    scratch_types=[
        pltpu.VMEM((CHUNK,), jnp.int32),
        pltpu.VMEM((CHUNK,), jnp.int32),
        pltpu.VMEM((CHUNK,), jnp.int32),
        pltpu.VMEM((CHUNK,), jnp.int32),
        pltpu.VMEM((CHUNK, D), jnp.float32),
        pltpu.VMEM((CHUNK, D), jnp.float32),
        pltpu.VMEM_SHARED((NPAD, D), jnp.float32),
        pltpu.SemaphoreType.DMA,
        pltpu.SemaphoreType.DMA,
        pltpu.SemaphoreType.DMA,
        pltpu.SemaphoreType.DMA,
    ],
)
def _sc_spmm(table_h, src_h, dst_h, zeros_h, out_h,
             srcb0, srcb1, dstb0, dstb1, buf0, buf1, acc_sh,
             semi0, semi1, semg0, semg1):
    c = lax.axis_index("c")
    s = lax.axis_index("s")
    wid = s * NC + c
    base = wid * CPT

    pltpu.sync_copy(zeros_h, acc_sh.at[pl.ds(s * ROWS_PER_TILE, ROWS_PER_TILE)])
    plsc.subcore_barrier()

    srcbs = (srcb0, srcb1)
    dstbs = (dstb0, dstb1)
    bufs = (buf0, buf1)
    semis = (semi0, semi1)
    semgs = (semg0, semg1)

    pltpu.sync_copy(src_h.at[base], srcb0)
    pltpu.sync_copy(dst_h.at[base], dstb0)
    pltpu.async_copy(src_h.at[base + 1], srcb1, semi1)
    pltpu.async_copy(dst_h.at[base + 1], dstb1, semi1)
    pltpu.async_copy(table_h.at[srcb0], buf0, semg0)

    def pair_body(g, carry):
        for b in range(2):
            i = g * 2 + b
            pltpu.make_async_copy(table_h.at[srcbs[b]], bufs[b],
                                  semgs[b]).wait()

            @pl.when(i + 1 < CPT)
            def _():
                pltpu.make_async_copy(src_h.at[base], srcbs[1 - b],
                                      semis[1 - b]).wait()
                pltpu.make_async_copy(dst_h.at[base], dstbs[1 - b],
                                      semis[1 - b]).wait()
                pltpu.async_copy(table_h.at[srcbs[1 - b]], bufs[1 - b],
                                 semgs[1 - b])

            pltpu.sync_copy(bufs[b], acc_sh.at[dstbs[b]], add=True)

            @pl.when(i + 2 < CPT)
            def _():
                pltpu.async_copy(src_h.at[base + i + 2], srcbs[b], semis[b])
                pltpu.async_copy(dst_h.at[base + i + 2], dstbs[b], semis[b])
        return carry

    lax.fori_loop(0, CPT // 2, pair_body, 0)
    plsc.subcore_barrier()

    pltpu.sync_copy(
        acc_sh.at[pl.ds(s * ROWS_PER_TILE, ROWS_PER_TILE)],
        out_h.at[c, pl.ds(s * ROWS_PER_TILE, ROWS_PER_TILE)],
    )


_ROWS_BLK = 256
_N_BLKS = NPAD // _ROWS_BLK


def _pre_body(xa_ref, w_ref, b_ref, x0_ref, h1_ref, h2_ref):
    xa = xa_ref[...]
    x0_ref[...] = jnp.dot(xa, w_ref[0], preferred_element_type=jnp.float32) + b_ref[0, 0]
    h1_ref[...] = jnp.dot(xa, w_ref[1], preferred_element_type=jnp.float32) + b_ref[0, 1]
    h2_ref[...] = jnp.dot(xa, w_ref[2], preferred_element_type=jnp.float32) + b_ref[0, 2]


def _tc_pre(xa, w_all, b_all):
    shp = jax.ShapeDtypeStruct((NPAD, D), jnp.float32)
    return pl.pallas_call(
        _pre_body,
        grid=(_N_BLKS,),
        in_specs=[
            pl.BlockSpec((_ROWS_BLK, D), lambda i: (i, 0)),
            pl.BlockSpec((3, D, D), lambda i: (0, 0, 0)),
            pl.BlockSpec((1, 3, D), lambda i: (0, 0, 0)),
        ],
        out_specs=[pl.BlockSpec((_ROWS_BLK, D), lambda i: (i, 0))] * 3,
        out_shape=[shp, shp, shp],
    )(xa, w_all, b_all)


def _add_body(a_ref, o_ref):
    o_ref[...] = a_ref[0] + a_ref[1]


def _tc_add(p):
    return pl.pallas_call(
        _add_body,
        grid=(_N_BLKS,),
        in_specs=[pl.BlockSpec((2, _ROWS_BLK, D), lambda i: (0, i, 0))],
        out_specs=pl.BlockSpec((_ROWS_BLK, D), lambda i: (i, 0)),
        out_shape=jax.ShapeDtypeStruct((NPAD, D), jnp.float32),
    )(p)


def _final_body(x0_ref, p1_ref, p3_ref, o_ref):
    o_ref[:, 0:D] = x0_ref[...]
    o_ref[:, D:2 * D] = p1_ref[0] + p1_ref[1]
    o_ref[:, 2 * D:3 * D] = p3_ref[0] + p3_ref[1]


def _tc_final(x0, p1, p3):
    return pl.pallas_call(
        _final_body,
        grid=(_N_BLKS,),
        in_specs=[
            pl.BlockSpec((_ROWS_BLK, D), lambda i: (i, 0)),
            pl.BlockSpec((2, _ROWS_BLK, D), lambda i: (0, i, 0)),
            pl.BlockSpec((2, _ROWS_BLK, D), lambda i: (0, i, 0)),
        ],
        out_specs=pl.BlockSpec((_ROWS_BLK, 3 * D), lambda i: (i, 0)),
        out_shape=jax.ShapeDtypeStruct((NPAD, 3 * D), jnp.float32),
    )(x0, p1, p3)


def kernel(x, edge_index, W0, b0, W1, b1, W2, b2):
    x = x.astype(jnp.float32)

    xa = jnp.zeros((NPAD, D), jnp.float32)
    xa = xa.at[:N_NODES].set(x)

    src = edge_index[0].astype(jnp.int32)
    dst = edge_index[1].astype(jnp.int32)
    pad = jnp.full((EPAD - N_EDGES,), N_NODES, jnp.int32)
    src2d = jnp.concatenate([src, pad]).reshape(NW * CPT, CHUNK)
    dst2d = jnp.concatenate([dst, pad]).reshape(NW * CPT, CHUNK)

    zeros = jnp.zeros((ROWS_PER_TILE, D), jnp.float32)

    w_all = jnp.stack([W0.T, W1.T, W2.T]).astype(jnp.float32)
    b_all = jnp.stack([b0, b1, b2]).astype(jnp.float32)[None]

    x0, h1, h2 = _tc_pre(xa, w_all, b_all)
    p1 = _sc_spmm(h1, src2d, dst2d, zeros)
    p2 = _sc_spmm(h2, src2d, dst2d, zeros)
    y2 = _tc_add(p2)
    p3 = _sc_spmm(y2, src2d, dst2d, zeros)
    out = _tc_final(x0, p1, p3)
    return out[:N_NODES]


# speedup vs baseline: 1.9233x; 1.9233x over previous
import functools

import jax
import jax.numpy as jnp
from jax import lax
from jax.experimental import pallas as pl
from jax.experimental.pallas import tpu as pltpu
from jax.experimental.pallas import tpu_sc as plsc

N_NODES = 10000
N_EDGES = 320000
D = 128
DH = D // 2
NPAD = 10240
NC = 2
NS = 16
CHUNK = 128
CPT = ((N_EDGES + NS * CHUNK - 1) // (NS * CHUNK) + 1) // 2 * 2
EPAD = NS * CPT * CHUNK
ROWS_PER_TILE = NPAD // NS

_mesh = plsc.VectorSubcoreMesh(core_axis_name="c", subcore_axis_name="s")


@functools.partial(
    pl.kernel,
    out_type=jax.ShapeDtypeStruct((NC, NPAD, DH), jnp.float32),
    mesh=_mesh,
    scratch_types=[
        pltpu.VMEM((CHUNK,), jnp.int32),
        pltpu.VMEM((CHUNK,), jnp.int32),
        pltpu.VMEM((CHUNK,), jnp.int32),
        pltpu.VMEM((CHUNK,), jnp.int32),
        pltpu.VMEM((CHUNK, DH), jnp.float32),
        pltpu.VMEM((CHUNK, DH), jnp.float32),
        pltpu.VMEM_SHARED((NPAD, DH), jnp.float32),
        pltpu.VMEM_SHARED((NPAD, DH), jnp.float32),
        pltpu.SemaphoreType.DMA,
        pltpu.SemaphoreType.DMA,
        pltpu.SemaphoreType.DMA,
        pltpu.SemaphoreType.DMA,
    ],
    compiler_params=pltpu.CompilerParams(use_tc_tiling_on_sc=False),
)
def _sc_spmm(table_h, src_h, dst_h, zeros_h, out_h,
             srcb0, srcb1, dstb0, dstb1, buf0, buf1, tab_sh, acc_sh,
             semi0, semi1, semg0, semg1):
    c = lax.axis_index("c")
    s = lax.axis_index("s")
    base = s * CPT
    stripe = pl.ds(s * ROWS_PER_TILE, ROWS_PER_TILE)

    pltpu.sync_copy(table_h.at[c, stripe], tab_sh.at[stripe])
    pltpu.sync_copy(zeros_h, acc_sh.at[stripe])
    plsc.subcore_barrier()

    srcbs = (srcb0, srcb1)
    dstbs = (dstb0, dstb1)
    bufs = (buf0, buf1)
    semis = (semi0, semi1)
    semgs = (semg0, semg1)

    pltpu.sync_copy(src_h.at[base], srcb0)
    pltpu.sync_copy(dst_h.at[base], dstb0)
    pltpu.async_copy(src_h.at[base + 1], srcb1, semi1)
    pltpu.async_copy(dst_h.at[base + 1], dstb1, semi1)
    pltpu.async_copy(tab_sh.at[srcb0], buf0, semg0)

    def pair_body(g, carry):
        for b in range(2):
            i = g * 2 + b
            pltpu.make_async_copy(tab_sh.at[srcbs[b]], bufs[b],
                                  semgs[b]).wait()

            @pl.when(i + 1 < CPT)
            def _():
                pltpu.make_async_copy(src_h.at[base], srcbs[1 - b],
                                      semis[1 - b]).wait()
                pltpu.make_async_copy(dst_h.at[base], dstbs[1 - b],
                                      semis[1 - b]).wait()
                pltpu.async_copy(tab_sh.at[srcbs[1 - b]], bufs[1 - b],
                                 semgs[1 - b])

            pltpu.sync_copy(bufs[b], acc_sh.at[dstbs[b]], add=True)

            @pl.when(i + 2 < CPT)
            def _():
                pltpu.async_copy(src_h.at[base + i + 2], srcbs[b], semis[b])
                pltpu.async_copy(dst_h.at[base + i + 2], dstbs[b], semis[b])
        return carry

    lax.fori_loop(0, CPT // 2, pair_body, 0)
    plsc.subcore_barrier()

    pltpu.sync_copy(acc_sh.at[stripe], out_h.at[c, stripe])


_ROWS_BLK = 256
_N_BLKS = NPAD // _ROWS_BLK


def _pre_body(xa_ref, w_ref, b_ref, x0_ref, h1_ref, h2_ref):
    xa = xa_ref[...]
    x0_ref[...] = jnp.dot(xa, w_ref[0], preferred_element_type=jnp.float32) + b_ref[0, 0]
    h1 = jnp.dot(xa, w_ref[1], preferred_element_type=jnp.float32) + b_ref[0, 1]
    h2 = jnp.dot(xa, w_ref[2], preferred_element_type=jnp.float32) + b_ref[0, 2]
    h1_ref[0] = h1[:, :DH]
    h1_ref[1] = h1[:, DH:]
    h2_ref[0] = h2[:, :DH]
    h2_ref[1] = h2[:, DH:]


def _tc_pre(xa, w_all, b_all):
    half = jax.ShapeDtypeStruct((NC, NPAD, DH), jnp.float32)
    return pl.pallas_call(
        _pre_body,
        grid=(_N_BLKS,),
        in_specs=[
            pl.BlockSpec((_ROWS_BLK, D), lambda i: (i, 0)),
            pl.BlockSpec((3, D, D), lambda i: (0, 0, 0)),
            pl.BlockSpec((1, 3, D), lambda i: (0, 0, 0)),
        ],
        out_specs=[
            pl.BlockSpec((_ROWS_BLK, D), lambda i: (i, 0)),
            pl.BlockSpec((NC, _ROWS_BLK, DH), lambda i: (0, i, 0)),
            pl.BlockSpec((NC, _ROWS_BLK, DH), lambda i: (0, i, 0)),
        ],
        out_shape=[jax.ShapeDtypeStruct((NPAD, D), jnp.float32), half, half],
    )(xa, w_all, b_all)


def _final_body(x0_ref, x1_ref, x2_ref, o_ref):
    o_ref[:, 0:D] = x0_ref[...]
    o_ref[:, D:D + DH] = x1_ref[0]
    o_ref[:, D + DH:2 * D] = x1_ref[1]
    o_ref[:, 2 * D:2 * D + DH] = x2_ref[0]
    o_ref[:, 2 * D + DH:3 * D] = x2_ref[1]


def _tc_final(x0, x1h, x2h):
    return pl.pallas_call(
        _final_body,
        grid=(_N_BLKS,),
        in_specs=[
            pl.BlockSpec((_ROWS_BLK, D), lambda i: (i, 0)),
            pl.BlockSpec((NC, _ROWS_BLK, DH), lambda i: (0, i, 0)),
            pl.BlockSpec((NC, _ROWS_BLK, DH), lambda i: (0, i, 0)),
        ],
        out_specs=pl.BlockSpec((_ROWS_BLK, 3 * D), lambda i: (i, 0)),
        out_shape=jax.ShapeDtypeStruct((NPAD, 3 * D), jnp.float32),
    )(x0, x1h, x2h)


def kernel(x, edge_index, W0, b0, W1, b1, W2, b2):
    x = x.astype(jnp.float32)

    xa = jnp.zeros((NPAD, D), jnp.float32)
    xa = xa.at[:N_NODES].set(x)

    src = edge_index[0].astype(jnp.int32)
    dst = edge_index[1].astype(jnp.int32)
    pad = jnp.full((EPAD - N_EDGES,), N_NODES, jnp.int32)
    src2d = jnp.concatenate([src, pad]).reshape(NS * CPT, CHUNK)
    dst2d = jnp.concatenate([dst, pad]).reshape(NS * CPT, CHUNK)

    zeros = jnp.zeros((ROWS_PER_TILE, DH), jnp.float32)

    w_all = jnp.stack([W0.T, W1.T, W2.T]).astype(jnp.float32)
    b_all = jnp.stack([b0, b1, b2]).astype(jnp.float32)[None]

    x0, h1h, h2h = _tc_pre(xa, w_all, b_all)
    x1h = _sc_spmm(h1h, src2d, dst2d, zeros)
    y2h = _sc_spmm(h2h, src2d, dst2d, zeros)
    x2h = _sc_spmm(y2h, src2d, dst2d, zeros)
    out = _tc_final(x0, x1h, x2h)
    return out[:N_NODES]


# speedup vs baseline: 2.6326x; 1.3688x over previous
import functools

import jax
import jax.numpy as jnp
from jax import lax
from jax.experimental import pallas as pl
from jax.experimental.pallas import tpu as pltpu
from jax.experimental.pallas import tpu_sc as plsc

N_NODES = 10000
N_EDGES = 320000
D = 128
DH = D // 2
NPAD = 10240
NC = 2
NS = 16
CHUNK = 128
CPT = ((N_EDGES + NS * CHUNK - 1) // (NS * CHUNK) + 1) // 2 * 2
EPAD = NS * CPT * CHUNK
ROWS_PER_TILE = NPAD // NS
DROWS = NPAD // DH
DSTRIPE = DROWS // NS

_mesh = plsc.VectorSubcoreMesh(core_axis_name="c", subcore_axis_name="s")
_params = pltpu.CompilerParams(use_tc_tiling_on_sc=False,
                               needs_layout_passes=False)

def _scratch(pass2):
    return [
        pltpu.VMEM((CHUNK,), jnp.int32),
        pltpu.VMEM((CHUNK,), jnp.int32),
        pltpu.VMEM((CHUNK,), jnp.int32),
        pltpu.VMEM((CHUNK,), jnp.int32),
        pltpu.VMEM((CHUNK, DH), jnp.float32),
        pltpu.VMEM((CHUNK, DH), jnp.float32),
        pltpu.VMEM((DROWS, DH), jnp.float32),
        pltpu.VMEM((DROWS // 2,), jnp.int32),
        pltpu.VMEM((DROWS // 2,), jnp.int32),
        pltpu.VMEM_SHARED((NPAD, DH), jnp.float32),
        pltpu.VMEM_SHARED((NPAD, DH), jnp.float32),
        pltpu.VMEM_SHARED((DROWS, DH), jnp.float32),
        pltpu.VMEM((DROWS, DH) if pass2 else (16,), jnp.float32),
        pltpu.SemaphoreType.DMA,
        pltpu.SemaphoreType.DMA,
        pltpu.SemaphoreType.DMA,
        pltpu.SemaphoreType.DMA,
    ]


def _sc_pass_body(pass2, table_h, src_h, dst_h, zeros_h, degin_h, out_h,
                  degout_h, srcb0, srcb1, dstb0, dstb1, buf0, buf1,
                  deg_v, idxa, idxb, tab_sh, acc_sh, deg_sh, degf_v,
                  semi0, semi1, semg0, semg1):
    c = lax.axis_index("c")
    s = lax.axis_index("s")
    base = s * CPT
    stripe = pl.ds(s * ROWS_PER_TILE, ROWS_PER_TILE)

    pltpu.sync_copy(table_h.at[c, stripe], tab_sh.at[stripe])
    pltpu.sync_copy(zeros_h, acc_sh.at[stripe])
    pltpu.sync_copy(zeros_h.at[pl.ds(0, DROWS)], deg_v)
    pltpu.sync_copy(zeros_h.at[pl.ds(0, DSTRIPE)],
                    deg_sh.at[pl.ds(s * DSTRIPE, DSTRIPE)])
    if pass2:
        pltpu.sync_copy(degin_h, degf_v)
    plsc.subcore_barrier()

    srcbs = (srcb0, srcb1)
    dstbs = (dstb0, dstb1)
    bufs = (buf0, buf1)
    semis = (semi0, semi1)
    semgs = (semg0, semg1)
    ones16 = jnp.ones((16,), jnp.float32)

    def deg_update(b):
        for j in range(CHUNK // 16):
            dd = dstbs[b][pl.ds(j * 16, 16)]
            rd = lax.shift_right_logical(dd, 6)
            cd = lax.bitwise_and(dd, 63)
            if pass2:
                ss = srcbs[b][pl.ds(j * 16, 16)]
                rs = lax.shift_right_logical(ss, 6)
                cs = lax.bitwise_and(ss, 63)
                val = plsc.load_gather(degf_v, [rs, cs])
            else:
                val = ones16
            plsc.addupdate_scatter(deg_v, [rd, cd], val)

    pltpu.sync_copy(src_h.at[base], srcb0)
    pltpu.sync_copy(dst_h.at[base], dstb0)
    pltpu.async_copy(src_h.at[base + 1], srcb1, semi1)
    pltpu.async_copy(dst_h.at[base + 1], dstb1, semi1)
    pltpu.async_copy(tab_sh.at[srcb0], buf0, semg0)

    def pair_body(g, carry):
        for b in range(2):
            i = g * 2 + b
            pltpu.make_async_copy(tab_sh.at[srcbs[b]], bufs[b],
                                  semgs[b]).wait()

            @pl.when(i + 1 < CPT)
            def _():
                pltpu.make_async_copy(src_h.at[base], srcbs[1 - b],
                                      semis[1 - b]).wait()
                pltpu.make_async_copy(dst_h.at[base], dstbs[1 - b],
                                      semis[1 - b]).wait()
                pltpu.async_copy(tab_sh.at[srcbs[1 - b]], bufs[1 - b],
                                 semgs[1 - b])

            pltpu.sync_copy(bufs[b], acc_sh.at[dstbs[b]], add=True)
            deg_update(b)

            @pl.when(i + 2 < CPT)
            def _():
                pltpu.async_copy(src_h.at[base + i + 2], srcbs[b], semis[b])
                pltpu.async_copy(dst_h.at[base + i + 2], dstbs[b], semis[b])
        return carry

    lax.fori_loop(0, CPT // 2, pair_body, 0)

    half = DROWS // 2
    iota16 = lax.iota(jnp.int32, 16)
    for k in range(half // 16):
        idxa[pl.ds(k * 16, 16)] = iota16 + (k * 16)
        idxb[pl.ds(k * 16, 16)] = iota16 + (half + k * 16)
    pltpu.sync_copy(deg_v.at[pl.ds(0, half)], deg_sh.at[idxa], add=True)
    pltpu.sync_copy(deg_v.at[pl.ds(half, half)], deg_sh.at[idxb], add=True)
    plsc.subcore_barrier()

    pltpu.sync_copy(acc_sh.at[stripe], out_h.at[c, stripe])

    @pl.when(c == 0)
    def _():
        pltpu.sync_copy(deg_sh.at[pl.ds(s * DSTRIPE, DSTRIPE)],
                        degout_h.at[pl.ds(s * DSTRIPE, DSTRIPE)])


_OUT_TYPES = (jax.ShapeDtypeStruct((NC, NPAD, DH), jnp.float32),
              jax.ShapeDtypeStruct((DROWS, DH), jnp.float32))

_sc_pass1 = functools.partial(
    pl.kernel, out_type=_OUT_TYPES, mesh=_mesh, compiler_params=_params,
    scratch_types=_scratch(False),
)(functools.partial(_sc_pass_body, False))

_sc_pass2 = functools.partial(
    pl.kernel, out_type=_OUT_TYPES, mesh=_mesh, compiler_params=_params,
    scratch_types=_scratch(True),
)(functools.partial(_sc_pass_body, True))


_ROWS_BLK = 256
_N_BLKS = NPAD // _ROWS_BLK


def _final_body(xa_ref, y1_ref, y2_ref, dg_ref, dg2_ref, w_ref, b_ref,
                o_ref):
    y1 = jnp.concatenate([y1_ref[0], y1_ref[1]], axis=1)
    y2 = jnp.concatenate([y2_ref[0], y2_ref[1]], axis=1)
    o_ref[:, 0:D] = (jnp.dot(xa_ref[...], w_ref[0],
                             preferred_element_type=jnp.float32)
                     + b_ref[0, 0])
    o_ref[:, D:2 * D] = (jnp.dot(y1, w_ref[1],
                                 preferred_element_type=jnp.float32)
                         + dg_ref[...] * b_ref[0, 1])
    o_ref[:, 2 * D:3 * D] = (jnp.dot(y2, w_ref[2],
                                     preferred_element_type=jnp.float32)
                             + dg2_ref[...] * b_ref[0, 2])


def _tc_final(xa, y1h, y2h, deg, deg2, w_all, b_all):
    return pl.pallas_call(
        _final_body,
        grid=(_N_BLKS,),
        in_specs=[
            pl.BlockSpec((_ROWS_BLK, D), lambda i: (i, 0)),
            pl.BlockSpec((NC, _ROWS_BLK, DH), lambda i: (0, i, 0)),
            pl.BlockSpec((NC, _ROWS_BLK, DH), lambda i: (0, i, 0)),
            pl.BlockSpec((_ROWS_BLK, 1), lambda i: (i, 0)),
            pl.BlockSpec((_ROWS_BLK, 1), lambda i: (i, 0)),
            pl.BlockSpec((3, D, D), lambda i: (0, 0, 0)),
            pl.BlockSpec((1, 3, D), lambda i: (0, 0, 0)),
        ],
        out_specs=pl.BlockSpec((_ROWS_BLK, 3 * D), lambda i: (i, 0)),
        out_shape=jax.ShapeDtypeStruct((NPAD, 3 * D), jnp.float32),
    )(xa, y1h, y2h, deg, deg2, w_all, b_all)


def kernel(x, edge_index, W0, b0, W1, b1, W2, b2):
    x = x.astype(jnp.float32)

    xa = jnp.zeros((NPAD, D), jnp.float32)
    xa = xa.at[:N_NODES].set(x)
    xh = jnp.stack([xa[:, :DH], xa[:, DH:]])

    src = edge_index[0].astype(jnp.int32)
    dst = edge_index[1].astype(jnp.int32)
    pad = jnp.full((EPAD - N_EDGES,), N_NODES, jnp.int32)
    src2d = jnp.concatenate([src, pad]).reshape(NS * CPT, CHUNK)
    dst2d = jnp.concatenate([dst, pad]).reshape(NS * CPT, CHUNK)

    zeros = jnp.zeros((ROWS_PER_TILE, DH), jnp.float32)
    dummy_deg = jnp.zeros((DROWS, DH), jnp.float32)

    w_all = jnp.stack([W0.T, W1.T, W2.T]).astype(jnp.float32)
    b_all = jnp.stack([b0, b1, b2]).astype(jnp.float32)[None]

    y1h, deg = _sc_pass1(xh, src2d, dst2d, zeros, dummy_deg)
    y2h, deg2 = _sc_pass2(y1h, src2d, dst2d, zeros, deg)
    out = _tc_final(xa, y1h, y2h, deg.reshape(NPAD, 1),
                    deg2.reshape(NPAD, 1), w_all, b_all)
    return out[:N_NODES]


# speedup vs baseline: 2.7234x; 1.0345x over previous
import functools

import jax
import jax.numpy as jnp
from jax import lax
from jax.experimental import pallas as pl
from jax.experimental.pallas import tpu as pltpu
from jax.experimental.pallas import tpu_sc as plsc

N_NODES = 10000
N_EDGES = 320000
D = 128
DH = D // 2
NPAD = 10240
NC = 2
NS = 16
CHUNK = 128
CPT = ((N_EDGES + NS * CHUNK - 1) // (NS * CHUNK) + 1) // 2 * 2
EPAD = NS * CPT * CHUNK
ROWS_PER_TILE = NPAD // NS
DROWS = NPAD // DH
DSTRIPE = DROWS // NS

_mesh = plsc.VectorSubcoreMesh(core_axis_name="c", subcore_axis_name="s")
_params = pltpu.CompilerParams(use_tc_tiling_on_sc=False,
                               needs_layout_passes=False)

def _scratch(pass2):
    return [
        pltpu.VMEM((CHUNK,), jnp.int32),
        pltpu.VMEM((CHUNK,), jnp.int32),
        pltpu.VMEM((CHUNK,), jnp.int32),
        pltpu.VMEM((CHUNK,), jnp.int32),
        pltpu.VMEM((CHUNK, DH), jnp.float32),
        pltpu.VMEM((CHUNK, DH), jnp.float32),
        pltpu.VMEM((DROWS, DH), jnp.float32),
        pltpu.VMEM((DROWS // 2,), jnp.int32),
        pltpu.VMEM((DROWS // 2,), jnp.int32),
        pltpu.VMEM_SHARED((NPAD, DH), jnp.float32),
        pltpu.VMEM_SHARED((NPAD, DH), jnp.float32),
        pltpu.VMEM_SHARED((DROWS, DH), jnp.float32),
        pltpu.VMEM((DROWS, DH) if pass2 else (16,), jnp.float32),
        pltpu.SemaphoreType.DMA,
        pltpu.SemaphoreType.DMA,
        pltpu.SemaphoreType.DMA,
        pltpu.SemaphoreType.DMA,
    ]


def _sc_pass_body(pass2, table_h, src_h, dst_h, zeros_h, degin_h, out_h,
                  degout_h, srcb0, srcb1, dstb0, dstb1, buf0, buf1,
                  deg_v, idxa, idxb, tab_sh, acc_sh, deg_sh, degf_v,
                  semi0, semi1, semg0, semg1):
    c = lax.axis_index("c")
    s = lax.axis_index("s")
    base = s * CPT
    stripe = pl.ds(s * ROWS_PER_TILE, ROWS_PER_TILE)

    pltpu.sync_copy(table_h.at[c, stripe], tab_sh.at[stripe])
    pltpu.sync_copy(zeros_h, acc_sh.at[stripe])
    pltpu.sync_copy(zeros_h.at[pl.ds(0, DROWS)], deg_v)
    pltpu.sync_copy(zeros_h.at[pl.ds(0, DSTRIPE)],
                    deg_sh.at[pl.ds(s * DSTRIPE, DSTRIPE)])
    if pass2:
        pltpu.sync_copy(degin_h, degf_v)
    plsc.subcore_barrier()

    srcbs = (srcb0, srcb1)
    dstbs = (dstb0, dstb1)
    bufs = (buf0, buf1)
    semis = (semi0, semi1)
    semgs = (semg0, semg1)
    ones16 = jnp.ones((16,), jnp.float32)

    def deg_update(b):
        for j in range(CHUNK // 16):
            dd = dstbs[b][pl.ds(j * 16, 16)]
            rd = lax.shift_right_logical(dd, 6)
            cd = lax.bitwise_and(dd, 63)
            if pass2:
                ss = srcbs[b][pl.ds(j * 16, 16)]
                rs = lax.shift_right_logical(ss, 6)
                cs = lax.bitwise_and(ss, 63)
                val = plsc.load_gather(degf_v, [rs, cs])
            else:
                val = ones16
            plsc.addupdate_scatter(deg_v, [rd, cd], val)

    pltpu.sync_copy(src_h.at[base], srcb0)
    pltpu.sync_copy(dst_h.at[base], dstb0)
    pltpu.async_copy(src_h.at[base + 1], srcb1, semi1)
    pltpu.async_copy(dst_h.at[base + 1], dstb1, semi1)
    pltpu.async_copy(tab_sh.at[srcb0], buf0, semg0)

    def pair_body(g, carry):
        for b in range(2):
            i = g * 2 + b
            pltpu.make_async_copy(tab_sh.at[srcbs[b]], bufs[b],
                                  semgs[b]).wait()

            @pl.when(i + 1 < CPT)
            def _():
                pltpu.make_async_copy(src_h.at[base], srcbs[1 - b],
                                      semis[1 - b]).wait()
                pltpu.make_async_copy(dst_h.at[base], dstbs[1 - b],
                                      semis[1 - b]).wait()
                pltpu.async_copy(tab_sh.at[srcbs[1 - b]], bufs[1 - b],
                                 semgs[1 - b])

            pltpu.sync_copy(bufs[b], acc_sh.at[dstbs[b]], add=True)
            deg_update(b)

            @pl.when(i + 2 < CPT)
            def _():
                pltpu.async_copy(src_h.at[base + i + 2], srcbs[b], semis[b])
                pltpu.async_copy(dst_h.at[base + i + 2], dstbs[b], semis[b])
        return carry

    lax.fori_loop(0, CPT // 2, pair_body, 0)

    half = DROWS // 2
    iota16 = lax.iota(jnp.int32, 16)
    for k in range(half // 16):
        idxa[pl.ds(k * 16, 16)] = iota16 + (k * 16)
        idxb[pl.ds(k * 16, 16)] = iota16 + (half + k * 16)
    pltpu.sync_copy(deg_v.at[pl.ds(0, half)], deg_sh.at[idxa], add=True)
    pltpu.sync_copy(deg_v.at[pl.ds(half, half)], deg_sh.at[idxb], add=True)
    plsc.subcore_barrier()

    pltpu.sync_copy(acc_sh.at[stripe], out_h.at[c, stripe])

    @pl.when(c == 0)
    def _():
        pltpu.sync_copy(deg_sh.at[pl.ds(s * DSTRIPE, DSTRIPE)],
                        degout_h.at[pl.ds(s * DSTRIPE, DSTRIPE)])


_OUT_TYPES = (jax.ShapeDtypeStruct((NC, NPAD, DH), jnp.float32),
              jax.ShapeDtypeStruct((DROWS, DH), jnp.float32))

_sc_pass1 = functools.partial(
    pl.kernel, out_type=_OUT_TYPES, mesh=_mesh, compiler_params=_params,
    scratch_types=_scratch(False),
)(functools.partial(_sc_pass_body, False))

_sc_pass2 = functools.partial(
    pl.kernel, out_type=_OUT_TYPES, mesh=_mesh, compiler_params=_params,
    scratch_types=_scratch(True),
)(functools.partial(_sc_pass_body, True))


_ROWS_BLK = 400
_N_BLKS = N_NODES // _ROWS_BLK


def _final_body(xa_ref, y1_ref, y2_ref, dg_ref, dg2_ref, w_ref, b_ref,
                o_ref):
    y1 = jnp.concatenate([y1_ref[0], y1_ref[1]], axis=1)
    y2 = jnp.concatenate([y2_ref[0], y2_ref[1]], axis=1)
    dg = dg_ref[...]
    dg2 = dg2_ref[...]
    o_ref[:, 0:D] = (jnp.dot(xa_ref[...], w_ref[0],
                             preferred_element_type=jnp.float32)
                     + b_ref[0, 0])
    o_ref[:, D:2 * D] = (jnp.dot(y1, w_ref[1],
                                 preferred_element_type=jnp.float32)
                         + dg * b_ref[0, 1])
    o_ref[:, 2 * D:3 * D] = (jnp.dot(y2, w_ref[2],
                                     preferred_element_type=jnp.float32)
                             + dg2 * b_ref[0, 2])


def _tc_final(xa, y1h, y2h, deg, deg2, w_all, b_all):
    return pl.pallas_call(
        _final_body,
        grid=(_N_BLKS,),
        in_specs=[
            pl.BlockSpec((_ROWS_BLK, D), lambda i: (i, 0)),
            pl.BlockSpec((NC, _ROWS_BLK, DH), lambda i: (0, i, 0)),
            pl.BlockSpec((NC, _ROWS_BLK, DH), lambda i: (0, i, 0)),
            pl.BlockSpec((_ROWS_BLK, 1), lambda i: (i, 0)),
            pl.BlockSpec((_ROWS_BLK, 1), lambda i: (i, 0)),
            pl.BlockSpec((3, D, D), lambda i: (0, 0, 0)),
            pl.BlockSpec((1, 3, D), lambda i: (0, 0, 0)),
        ],
        out_specs=pl.BlockSpec((_ROWS_BLK, 3 * D), lambda i: (i, 0)),
        out_shape=jax.ShapeDtypeStruct((N_NODES, 3 * D), jnp.float32),
    )(xa, y1h, y2h, deg, deg2, w_all, b_all)


def kernel(x, edge_index, W0, b0, W1, b1, W2, b2):
    x = x.astype(jnp.float32)

    xa = jnp.zeros((NPAD, D), jnp.float32)
    xa = xa.at[:N_NODES].set(x)
    xh = jnp.stack([xa[:, :DH], xa[:, DH:]])

    src = edge_index[0].astype(jnp.int32)
    dst = edge_index[1].astype(jnp.int32)
    pad = N_NODES + jnp.arange(EPAD - N_EDGES, dtype=jnp.int32) % (NPAD - N_NODES)
    src2d = jnp.concatenate([src, pad]).reshape(NS * CPT, CHUNK)
    dst2d = jnp.concatenate([dst, pad]).reshape(NS * CPT, CHUNK)

    zeros = jnp.zeros((ROWS_PER_TILE, DH), jnp.float32)
    dummy_deg = jnp.zeros((DROWS, DH), jnp.float32)

    w_all = jnp.stack([W0.T, W1.T, W2.T]).astype(jnp.float32)
    b_all = jnp.stack([b0, b1, b2]).astype(jnp.float32)[None]

    y1h, deg = _sc_pass1(xh, src2d, dst2d, zeros, dummy_deg)
    y2h, deg2 = _sc_pass2(y1h, src2d, dst2d, zeros, deg)
    return _tc_final(xa, y1h, y2h, deg.reshape(NPAD, 1),
                     deg2.reshape(NPAD, 1), w_all, b_all)


# speedup vs baseline: 3.6454x; 1.3386x over previous
import functools

import jax
import jax.numpy as jnp
from jax import lax
from jax.experimental import pallas as pl
from jax.experimental.pallas import tpu as pltpu
from jax.experimental.pallas import tpu_sc as plsc

N_NODES = 10000
N_EDGES = 320000
D = 128
DH = D // 2
NPAD = 10240
NC = 2
NS = 16
CHUNK = 128
NBUF = 3
CPT = ((N_EDGES + NS * CHUNK - 1) // (NS * CHUNK) + NBUF - 1) // NBUF * NBUF
EPAD = NS * CPT * CHUNK
ROWS_PER_TILE = NPAD // NS
DROWS = NPAD // DH
DSTRIPE = DROWS // NS

_mesh = plsc.VectorSubcoreMesh(core_axis_name="c", subcore_axis_name="s")
_params = pltpu.CompilerParams(use_tc_tiling_on_sc=False,
                               needs_layout_passes=False)

def _scratch(pass2):
    return (
        [pltpu.VMEM((CHUNK,), jnp.int32) for _ in range(NBUF)]
        + [pltpu.VMEM((CHUNK,), jnp.int32) for _ in range(NBUF)]
        + [pltpu.VMEM((CHUNK, DH), jnp.float32) for _ in range(NBUF)]
        + [
            pltpu.VMEM((DROWS, DH), jnp.float32),
            pltpu.VMEM((DROWS // 2,), jnp.int32),
            pltpu.VMEM((DROWS // 2,), jnp.int32),
            pltpu.VMEM_SHARED((NPAD, DH), jnp.float32),
            pltpu.VMEM_SHARED((NPAD, DH), jnp.float32),
            pltpu.VMEM_SHARED((DROWS, DH), jnp.float32),
            pltpu.VMEM((DROWS, DH) if pass2 else (16,), jnp.float32),
        ]
        + [pltpu.SemaphoreType.DMA for _ in range(3 * NBUF)]
    )


def _sc_pass_body(pass2, table_h, src_h, dst_h, zeros_h, degin_h, out_h,
                  degout_h, *refs):
    srcbs = refs[0:NBUF]
    dstbs = refs[NBUF:2 * NBUF]
    bufs = refs[2 * NBUF:3 * NBUF]
    deg_v, idxa, idxb, tab_sh, acc_sh, deg_sh, degf_v = refs[3 * NBUF:
                                                            3 * NBUF + 7]
    semis = refs[3 * NBUF + 7:3 * NBUF + 7 + NBUF]
    semgs = refs[3 * NBUF + 7 + NBUF:3 * NBUF + 7 + 2 * NBUF]
    semss = refs[3 * NBUF + 7 + 2 * NBUF:3 * NBUF + 7 + 3 * NBUF]
    c = lax.axis_index("c")
    s = lax.axis_index("s")
    base = s * CPT
    stripe = pl.ds(s * ROWS_PER_TILE, ROWS_PER_TILE)

    pltpu.sync_copy(table_h.at[c, stripe], tab_sh.at[stripe])
    pltpu.sync_copy(zeros_h, acc_sh.at[stripe])
    pltpu.sync_copy(zeros_h.at[pl.ds(0, DROWS)], deg_v)
    pltpu.sync_copy(zeros_h.at[pl.ds(0, DSTRIPE)],
                    deg_sh.at[pl.ds(s * DSTRIPE, DSTRIPE)])
    if pass2:
        pltpu.sync_copy(degin_h, degf_v)
    plsc.subcore_barrier()

    ones16 = jnp.ones((16,), jnp.float32)

    def deg_update(b):
        for j in range(CHUNK // 16):
            dd = dstbs[b][pl.ds(j * 16, 16)]
            rd = lax.shift_right_logical(dd, 6)
            cd = lax.bitwise_and(dd, 63)
            if pass2:
                ss = srcbs[b][pl.ds(j * 16, 16)]
                rs = lax.shift_right_logical(ss, 6)
                cs = lax.bitwise_and(ss, 63)
                val = plsc.load_gather(degf_v, [rs, cs])
            else:
                val = ones16
            plsc.addupdate_scatter(deg_v, [rd, cd], val)

    pltpu.sync_copy(src_h.at[base], srcbs[0])
    pltpu.sync_copy(dst_h.at[base], dstbs[0])
    pltpu.async_copy(src_h.at[base + 1], srcbs[1], semis[1])
    pltpu.async_copy(dst_h.at[base + 1], dstbs[1], semis[1])
    pltpu.async_copy(tab_sh.at[srcbs[0]], bufs[0], semgs[0])

    def tri_body(g, carry):
        for b in range(NBUF):
            i = g * NBUF + b
            p = (b + 2) % NBUF
            n = (b + 1) % NBUF

            @pl.when(i >= 1)
            def _():
                pltpu.make_async_copy(bufs[p], acc_sh.at[dstbs[p]],
                                      semss[p]).wait()

            @pl.when(i + 2 < CPT)
            def _():
                pltpu.async_copy(src_h.at[base + i + 2], srcbs[p], semis[p])
                pltpu.async_copy(dst_h.at[base + i + 2], dstbs[p], semis[p])

            pltpu.make_async_copy(tab_sh.at[srcbs[b]], bufs[b],
                                  semgs[b]).wait()
            pltpu.async_copy(bufs[b], acc_sh.at[dstbs[b]], semss[b],
                             add=True)

            @pl.when(i + 1 < CPT)
            def _():
                pltpu.make_async_copy(src_h.at[base], srcbs[n],
                                      semis[n]).wait()
                pltpu.make_async_copy(dst_h.at[base], dstbs[n],
                                      semis[n]).wait()
                pltpu.async_copy(tab_sh.at[srcbs[n]], bufs[n], semgs[n])

            deg_update(b)
        return carry

    lax.fori_loop(0, CPT // NBUF, tri_body, 0)
    pltpu.make_async_copy(bufs[(CPT - 1) % NBUF],
                          acc_sh.at[dstbs[(CPT - 1) % NBUF]],
                          semss[(CPT - 1) % NBUF]).wait()

    half = DROWS // 2
    iota16 = lax.iota(jnp.int32, 16)
    for k in range(half // 16):
        idxa[pl.ds(k * 16, 16)] = iota16 + (k * 16)
        idxb[pl.ds(k * 16, 16)] = iota16 + (half + k * 16)
    pltpu.sync_copy(deg_v.at[pl.ds(0, half)], deg_sh.at[idxa], add=True)
    pltpu.sync_copy(deg_v.at[pl.ds(half, half)], deg_sh.at[idxb], add=True)
    plsc.subcore_barrier()

    pltpu.sync_copy(acc_sh.at[stripe], out_h.at[c, stripe])

    @pl.when(c == 0)
    def _():
        pltpu.sync_copy(deg_sh.at[pl.ds(s * DSTRIPE, DSTRIPE)],
                        degout_h.at[pl.ds(s * DSTRIPE, DSTRIPE)])


_OUT_TYPES = (jax.ShapeDtypeStruct((NC, NPAD, DH), jnp.float32),
              jax.ShapeDtypeStruct((DROWS, DH), jnp.float32))

_sc_pass1 = functools.partial(
    pl.kernel, out_type=_OUT_TYPES, mesh=_mesh, compiler_params=_params,
    scratch_types=_scratch(False),
)(functools.partial(_sc_pass_body, False))

_sc_pass2 = functools.partial(
    pl.kernel, out_type=_OUT_TYPES, mesh=_mesh, compiler_params=_params,
    scratch_types=_scratch(True),
)(functools.partial(_sc_pass_body, True))


_ROWS_BLK = 400
_N_BLKS = N_NODES // _ROWS_BLK


def _final_body(xa_ref, y1_ref, y2_ref, dg_ref, dg2_ref, w_ref, b_ref,
                o_ref):
    y1 = jnp.concatenate([y1_ref[0], y1_ref[1]], axis=1)
    y2 = jnp.concatenate([y2_ref[0], y2_ref[1]], axis=1)
    dg = dg_ref[...]
    dg2 = dg2_ref[...]
    o_ref[:, 0:D] = (jnp.dot(xa_ref[...], w_ref[0],
                             preferred_element_type=jnp.float32)
                     + b_ref[0, 0])
    o_ref[:, D:2 * D] = (jnp.dot(y1, w_ref[1],
                                 preferred_element_type=jnp.float32)
                         + dg * b_ref[0, 1])
    o_ref[:, 2 * D:3 * D] = (jnp.dot(y2, w_ref[2],
                                     preferred_element_type=jnp.float32)
                             + dg2 * b_ref[0, 2])


def _tc_final(xa, y1h, y2h, deg, deg2, w_all, b_all):
    return pl.pallas_call(
        _final_body,
        grid=(_N_BLKS,),
        in_specs=[
            pl.BlockSpec((_ROWS_BLK, D), lambda i: (i, 0)),
            pl.BlockSpec((NC, _ROWS_BLK, DH), lambda i: (0, i, 0)),
            pl.BlockSpec((NC, _ROWS_BLK, DH), lambda i: (0, i, 0)),
            pl.BlockSpec((_ROWS_BLK, 1), lambda i: (i, 0)),
            pl.BlockSpec((_ROWS_BLK, 1), lambda i: (i, 0)),
            pl.BlockSpec((3, D, D), lambda i: (0, 0, 0)),
            pl.BlockSpec((1, 3, D), lambda i: (0, 0, 0)),
        ],
        out_specs=pl.BlockSpec((_ROWS_BLK, 3 * D), lambda i: (i, 0)),
        out_shape=jax.ShapeDtypeStruct((N_NODES, 3 * D), jnp.float32),
    )(xa, y1h, y2h, deg, deg2, w_all, b_all)


def kernel(x, edge_index, W0, b0, W1, b1, W2, b2):
    x = x.astype(jnp.float32)

    xa = jnp.zeros((NPAD, D), jnp.float32)
    xa = xa.at[:N_NODES].set(x)
    xh = jnp.stack([xa[:, :DH], xa[:, DH:]])

    src = edge_index[0].astype(jnp.int32)
    dst = edge_index[1].astype(jnp.int32)
    pad = N_NODES + jnp.arange(EPAD - N_EDGES, dtype=jnp.int32) % (NPAD - N_NODES)
    src2d = jnp.concatenate([src, pad]).reshape(NS * CPT, CHUNK)
    dst2d = jnp.concatenate([dst, pad]).reshape(NS * CPT, CHUNK)

    zeros = jnp.zeros((ROWS_PER_TILE, DH), jnp.float32)
    dummy_deg = jnp.zeros((DROWS, DH), jnp.float32)

    w_all = jnp.stack([W0.T, W1.T, W2.T]).astype(jnp.float32)
    b_all = jnp.stack([b0, b1, b2]).astype(jnp.float32)[None]

    y1h, deg = _sc_pass1(xh, src2d, dst2d, zeros, dummy_deg)
    y2h, deg2 = _sc_pass2(y1h, src2d, dst2d, zeros, deg)
    return _tc_final(xa, y1h, y2h, deg.reshape(NPAD, 1),
                     deg2.reshape(NPAD, 1), w_all, b_all)


# speedup vs baseline: 3.8528x; 1.0569x over previous
import functools

import jax
import jax.numpy as jnp
from jax import lax
from jax.experimental import pallas as pl
from jax.experimental.pallas import tpu as pltpu
from jax.experimental.pallas import tpu_sc as plsc

N_NODES = 10000
N_EDGES = 320000
D = 128
DH = D // 2
NPAD = 10240
NC = 2
NS = 16
CHUNK = 128
NBUF = 3
CPT = ((N_EDGES + NS * CHUNK - 1) // (NS * CHUNK) + NBUF - 1) // NBUF * NBUF
EPAD = NS * CPT * CHUNK
ROWS_PER_TILE = NPAD // NS
DROWS = NPAD // DH
DSTRIPE = DROWS // NS

_mesh = plsc.VectorSubcoreMesh(core_axis_name="c", subcore_axis_name="s")
_params = pltpu.CompilerParams(use_tc_tiling_on_sc=False,
                               needs_layout_passes=False)

def _scratch(pass2):
    return (
        [pltpu.VMEM((CHUNK,), jnp.int32) for _ in range(NBUF)]
        + [pltpu.VMEM((CHUNK,), jnp.int32) for _ in range(NBUF)]
        + [pltpu.VMEM((CHUNK, DH), jnp.float32) for _ in range(NBUF)]
        + [
            pltpu.VMEM((DROWS, DH), jnp.float32),
            pltpu.VMEM((DROWS // 2,), jnp.int32),
            pltpu.VMEM((DROWS // 2,), jnp.int32),
            pltpu.VMEM_SHARED((NPAD, DH), jnp.float32),
            pltpu.VMEM_SHARED((NPAD, DH), jnp.float32),
            pltpu.VMEM_SHARED((DROWS, DH), jnp.float32),
            pltpu.VMEM((DROWS, DH) if pass2 else (16,), jnp.float32),
        ]
        + [pltpu.SemaphoreType.DMA for _ in range(3 * NBUF)]
    )


def _sc_pass_body(pass2, table_h, src_h, dst_h, zeros_h, degin_h, out_h,
                  degout_h, *refs):
    srcbs = refs[0:NBUF]
    dstbs = refs[NBUF:2 * NBUF]
    bufs = refs[2 * NBUF:3 * NBUF]
    deg_v, idxa, idxb, tab_sh, acc_sh, deg_sh, degf_v = refs[3 * NBUF:
                                                            3 * NBUF + 7]
    semis = refs[3 * NBUF + 7:3 * NBUF + 7 + NBUF]
    semgs = refs[3 * NBUF + 7 + NBUF:3 * NBUF + 7 + 2 * NBUF]
    semss = refs[3 * NBUF + 7 + 2 * NBUF:3 * NBUF + 7 + 3 * NBUF]
    c = lax.axis_index("c")
    s = lax.axis_index("s")
    base = s * CPT
    stripe = pl.ds(s * ROWS_PER_TILE, ROWS_PER_TILE)

    if pass2:
        pltpu.sync_copy(table_h.at[c, stripe], tab_sh.at[stripe])
    else:
        nrpt = N_NODES // NS
        pltpu.sync_copy(
            table_h.at[pl.ds(s * nrpt, nrpt), pl.ds(c * DH, DH)],
            tab_sh.at[pl.ds(s * nrpt, nrpt)])
    pltpu.sync_copy(zeros_h, acc_sh.at[stripe])
    pltpu.sync_copy(zeros_h.at[pl.ds(0, DROWS)], deg_v)
    pltpu.sync_copy(zeros_h.at[pl.ds(0, DSTRIPE)],
                    deg_sh.at[pl.ds(s * DSTRIPE, DSTRIPE)])
    if pass2:
        pltpu.sync_copy(degin_h, degf_v)
    plsc.subcore_barrier()

    ones16 = jnp.ones((16,), jnp.float32)

    def deg_update(b):
        for j in range(CHUNK // 16):
            dd = dstbs[b][pl.ds(j * 16, 16)]
            rd = lax.shift_right_logical(dd, 6)
            cd = lax.bitwise_and(dd, 63)
            if pass2:
                ss = srcbs[b][pl.ds(j * 16, 16)]
                rs = lax.shift_right_logical(ss, 6)
                cs = lax.bitwise_and(ss, 63)
                val = plsc.load_gather(degf_v, [rs, cs])
            else:
                val = ones16
            plsc.addupdate_scatter(deg_v, [rd, cd], val)

    pltpu.sync_copy(src_h.at[base], srcbs[0])
    pltpu.sync_copy(dst_h.at[base], dstbs[0])
    pltpu.async_copy(src_h.at[base + 1], srcbs[1], semis[1])
    pltpu.async_copy(dst_h.at[base + 1], dstbs[1], semis[1])
    pltpu.async_copy(tab_sh.at[srcbs[0]], bufs[0], semgs[0])

    def tri_body(g, carry):
        for b in range(NBUF):
            i = g * NBUF + b
            p = (b + 2) % NBUF
            n = (b + 1) % NBUF

            @pl.when(i >= 1)
            def _():
                pltpu.make_async_copy(bufs[p], acc_sh.at[dstbs[p]],
                                      semss[p]).wait()

            @pl.when(i + 2 < CPT)
            def _():
                pltpu.async_copy(src_h.at[base + i + 2], srcbs[p], semis[p])
                pltpu.async_copy(dst_h.at[base + i + 2], dstbs[p], semis[p])

            pltpu.make_async_copy(tab_sh.at[srcbs[b]], bufs[b],
                                  semgs[b]).wait()
            pltpu.async_copy(bufs[b], acc_sh.at[dstbs[b]], semss[b],
                             add=True)

            @pl.when(i + 1 < CPT)
            def _():
                pltpu.make_async_copy(src_h.at[base], srcbs[n],
                                      semis[n]).wait()
                pltpu.make_async_copy(dst_h.at[base], dstbs[n],
                                      semis[n]).wait()
                pltpu.async_copy(tab_sh.at[srcbs[n]], bufs[n], semgs[n])

            deg_update(b)
        return carry

    lax.fori_loop(0, CPT // NBUF, tri_body, 0)
    pltpu.make_async_copy(bufs[(CPT - 1) % NBUF],
                          acc_sh.at[dstbs[(CPT - 1) % NBUF]],
                          semss[(CPT - 1) % NBUF]).wait()

    half = DROWS // 2
    iota16 = lax.iota(jnp.int32, 16)
    for k in range(half // 16):
        idxa[pl.ds(k * 16, 16)] = iota16 + (k * 16)
        idxb[pl.ds(k * 16, 16)] = iota16 + (half + k * 16)
    pltpu.sync_copy(deg_v.at[pl.ds(0, half)], deg_sh.at[idxa], add=True)
    pltpu.sync_copy(deg_v.at[pl.ds(half, half)], deg_sh.at[idxb], add=True)
    plsc.subcore_barrier()

    pltpu.sync_copy(acc_sh.at[stripe], out_h.at[c, stripe])

    @pl.when(c == 0)
    def _():
        pltpu.sync_copy(deg_sh.at[pl.ds(s * DSTRIPE, DSTRIPE)],
                        degout_h.at[pl.ds(s * DSTRIPE, DSTRIPE)])


_OUT_TYPES = (jax.ShapeDtypeStruct((NC, NPAD, DH), jnp.float32),
              jax.ShapeDtypeStruct((DROWS, DH), jnp.float32))

_sc_pass1 = functools.partial(
    pl.kernel, out_type=_OUT_TYPES, mesh=_mesh, compiler_params=_params,
    scratch_types=_scratch(False),
)(functools.partial(_sc_pass_body, False))

_sc_pass2 = functools.partial(
    pl.kernel, out_type=_OUT_TYPES, mesh=_mesh, compiler_params=_params,
    scratch_types=_scratch(True),
)(functools.partial(_sc_pass_body, True))


_ROWS_BLK = 400
_N_BLKS = N_NODES // _ROWS_BLK


def _final01_body(x_ref, y1_ref, dg_ref, w_ref, b_ref, o_ref):
    y1 = jnp.concatenate([y1_ref[0], y1_ref[1]], axis=1)
    o_ref[:, 0:D] = (jnp.dot(x_ref[...], w_ref[0],
                             preferred_element_type=jnp.float32)
                     + b_ref[0, 0])
    o_ref[:, D:2 * D] = (jnp.dot(y1, w_ref[1],
                                 preferred_element_type=jnp.float32)
                         + dg_ref[...] * b_ref[0, 1])


def _tc_final01(x, y1h, deg, w_all, b_all):
    return pl.pallas_call(
        _final01_body,
        grid=(_N_BLKS,),
        in_specs=[
            pl.BlockSpec((_ROWS_BLK, D), lambda i: (i, 0)),
            pl.BlockSpec((NC, _ROWS_BLK, DH), lambda i: (0, i, 0)),
            pl.BlockSpec((_ROWS_BLK, 1), lambda i: (i, 0)),
            pl.BlockSpec((3, D, D), lambda i: (0, 0, 0)),
            pl.BlockSpec((1, 3, D), lambda i: (0, 0, 0)),
        ],
        out_specs=pl.BlockSpec((_ROWS_BLK, 3 * D), lambda i: (i, 0)),
        out_shape=jax.ShapeDtypeStruct((N_NODES, 3 * D), jnp.float32),
    )(x, y1h, deg, w_all, b_all)


def _final2_body(y2_ref, dg2_ref, w_ref, b_ref, out01_ref, o_ref):
    del out01_ref
    y2 = jnp.concatenate([y2_ref[0], y2_ref[1]], axis=1)
    o_ref[...] = (jnp.dot(y2, w_ref[2], preferred_element_type=jnp.float32)
                  + dg2_ref[...] * b_ref[0, 2])


def _tc_final2(y2h, deg2, w_all, b_all, out01):
    return pl.pallas_call(
        _final2_body,
        grid=(_N_BLKS,),
        in_specs=[
            pl.BlockSpec((NC, _ROWS_BLK, DH), lambda i: (0, i, 0)),
            pl.BlockSpec((_ROWS_BLK, 1), lambda i: (i, 0)),
            pl.BlockSpec((3, D, D), lambda i: (0, 0, 0)),
            pl.BlockSpec((1, 3, D), lambda i: (0, 0, 0)),
            pl.BlockSpec(memory_space=pl.ANY),
        ],
        out_specs=pl.BlockSpec((_ROWS_BLK, D), lambda i: (i, 2)),
        out_shape=jax.ShapeDtypeStruct((N_NODES, 3 * D), jnp.float32),
        input_output_aliases={4: 0},
    )(y2h, deg2, w_all, b_all, out01)


def kernel(x, edge_index, W0, b0, W1, b1, W2, b2):
    x = x.astype(jnp.float32)

    src = edge_index[0].astype(jnp.int32)
    dst = edge_index[1].astype(jnp.int32)
    npad_e = EPAD - N_EDGES
    pad_dst = N_NODES + jnp.arange(npad_e, dtype=jnp.int32) % (NPAD - N_NODES)
    src2d = jnp.concatenate([src, jnp.zeros((npad_e,), jnp.int32)]
                            ).reshape(NS * CPT, CHUNK)
    dst2d = jnp.concatenate([dst, pad_dst]).reshape(NS * CPT, CHUNK)

    zeros = jnp.zeros((ROWS_PER_TILE, DH), jnp.float32)
    dummy_deg = jnp.zeros((DROWS, DH), jnp.float32)

    w_all = jnp.stack([W0.T, W1.T, W2.T]).astype(jnp.float32)
    b_all = jnp.stack([b0, b1, b2]).astype(jnp.float32)[None]

    y1h, deg = _sc_pass1(x, src2d, dst2d, zeros, dummy_deg)
    y2h, deg2 = _sc_pass2(y1h, src2d, dst2d, zeros, deg)
    out01 = _tc_final01(x, y1h, deg.reshape(NPAD, 1), w_all, b_all)
    return _tc_final2(y2h, deg2.reshape(NPAD, 1), w_all, b_all, out01)


# speedup vs baseline: 3.8545x; 1.0004x over previous
import functools

import jax
import jax.numpy as jnp
from jax import lax
from jax.experimental import pallas as pl
from jax.experimental.pallas import tpu as pltpu
from jax.experimental.pallas import tpu_sc as plsc

N_NODES = 10000
N_EDGES = 320000
D = 128
DH = D // 2
NPAD = 10240
NC = 2
NS = 16
CHUNK = 128
NBUF = 3
CPT = ((N_EDGES + NS * CHUNK - 1) // (NS * CHUNK) + NBUF - 1) // NBUF * NBUF
EPAD = NS * CPT * CHUNK
ROWS_PER_TILE = NPAD // NS
DROWS = NPAD // DH
DSTRIPE = DROWS // NS

_mesh = plsc.VectorSubcoreMesh(core_axis_name="c", subcore_axis_name="s")
_params = pltpu.CompilerParams(use_tc_tiling_on_sc=False,
                               needs_layout_passes=False)

def _scratch(pass2):
    return (
        [pltpu.VMEM((CHUNK,), jnp.int32) for _ in range(NBUF)]
        + [pltpu.VMEM((CHUNK,), jnp.int32) for _ in range(NBUF)]
        + [pltpu.VMEM((CHUNK, DH), jnp.float32) for _ in range(NBUF)]
        + [
            pltpu.VMEM((DROWS, DH), jnp.float32),
            pltpu.VMEM((DROWS // 2,), jnp.int32),
            pltpu.VMEM((DROWS // 2,), jnp.int32),
            pltpu.VMEM_SHARED((NPAD, DH), jnp.float32),
            pltpu.VMEM_SHARED((NPAD, DH), jnp.float32),
            pltpu.VMEM_SHARED((DROWS, DH), jnp.float32),
            pltpu.VMEM((DROWS, DH) if pass2 else (16,), jnp.float32),
        ]
        + [pltpu.SemaphoreType.DMA for _ in range(3 * NBUF)]
    )


def _sc_pass_body(pass2, table_h, src_h, dst_h, zeros_h, degin_h, out_h,
                  degout_h, *refs):
    srcbs = refs[0:NBUF]
    dstbs = refs[NBUF:2 * NBUF]
    bufs = refs[2 * NBUF:3 * NBUF]
    deg_v, idxa, idxb, tab_sh, acc_sh, deg_sh, degf_v = refs[3 * NBUF:
                                                            3 * NBUF + 7]
    semis = refs[3 * NBUF + 7:3 * NBUF + 7 + NBUF]
    semgs = refs[3 * NBUF + 7 + NBUF:3 * NBUF + 7 + 2 * NBUF]
    semss = refs[3 * NBUF + 7 + 2 * NBUF:3 * NBUF + 7 + 3 * NBUF]
    c = lax.axis_index("c")
    s = lax.axis_index("s")
    base = s * CPT
    stripe = pl.ds(s * ROWS_PER_TILE, ROWS_PER_TILE)

    if pass2:
        pltpu.sync_copy(table_h.at[c, stripe], tab_sh.at[stripe])
    else:
        nrpt = N_NODES // NS
        pltpu.sync_copy(
            table_h.at[pl.ds(s * nrpt, nrpt), pl.ds(c * DH, DH)],
            tab_sh.at[pl.ds(s * nrpt, nrpt)])
    pltpu.sync_copy(zeros_h, acc_sh.at[stripe])
    pltpu.sync_copy(zeros_h.at[pl.ds(0, DROWS)], deg_v)
    pltpu.sync_copy(zeros_h.at[pl.ds(0, DSTRIPE)],
                    deg_sh.at[pl.ds(s * DSTRIPE, DSTRIPE)])
    if pass2:
        pltpu.sync_copy(degin_h, degf_v)
    plsc.subcore_barrier()

    ones16 = jnp.ones((16,), jnp.float32)

    def deg_update(b):
        for j in range(CHUNK // 16):
            dd = dstbs[b][pl.ds(j * 16, 16)]
            rd = lax.shift_right_logical(dd, 6)
            cd = lax.bitwise_and(dd, 63)
            if pass2:
                ss = srcbs[b][pl.ds(j * 16, 16)]
                rs = lax.shift_right_logical(ss, 6)
                cs = lax.bitwise_and(ss, 63)
                val = plsc.load_gather(degf_v, [rs, cs])
            else:
                val = ones16
            plsc.addupdate_scatter(deg_v, [rd, cd], val)

    pltpu.sync_copy(src_h.at[base], srcbs[0])
    pltpu.sync_copy(dst_h.at[base], dstbs[0])
    pltpu.async_copy(src_h.at[base + 1], srcbs[1], semis[1])
    pltpu.async_copy(dst_h.at[base + 1], dstbs[1], semis[1])
    pltpu.async_copy(tab_sh.at[srcbs[0]], bufs[0], semgs[0])

    def tri_body(g, carry):
        for b in range(NBUF):
            i = g * NBUF + b
            p = (b + 2) % NBUF
            n = (b + 1) % NBUF

            @pl.when(i >= 1)
            def _():
                pltpu.make_async_copy(bufs[p], acc_sh.at[dstbs[p]],
                                      semss[p]).wait()

            @pl.when(i + 2 < CPT)
            def _():
                pltpu.async_copy(src_h.at[base + i + 2], srcbs[p], semis[p])
                pltpu.async_copy(dst_h.at[base + i + 2], dstbs[p], semis[p])

            pltpu.make_async_copy(tab_sh.at[srcbs[b]], bufs[b],
                                  semgs[b]).wait()
            pltpu.async_copy(bufs[b], acc_sh.at[dstbs[b]], semss[b],
                             add=True)

            @pl.when(i + 1 < CPT)
            def _():
                pltpu.make_async_copy(src_h.at[base], srcbs[n],
                                      semis[n]).wait()
                pltpu.make_async_copy(dst_h.at[base], dstbs[n],
                                      semis[n]).wait()
                pltpu.async_copy(tab_sh.at[srcbs[n]], bufs[n], semgs[n])

            deg_update(b)
        return carry

    lax.fori_loop(0, CPT // NBUF, tri_body, 0)
    pltpu.make_async_copy(bufs[(CPT - 1) % NBUF],
                          acc_sh.at[dstbs[(CPT - 1) % NBUF]],
                          semss[(CPT - 1) % NBUF]).wait()

    half = DROWS // 2
    iota16 = lax.iota(jnp.int32, 16)
    for k in range(half // 16):
        idxa[pl.ds(k * 16, 16)] = iota16 + (k * 16)
        idxb[pl.ds(k * 16, 16)] = iota16 + (half + k * 16)
    pltpu.sync_copy(deg_v.at[pl.ds(0, half)], deg_sh.at[idxa], add=True)
    pltpu.sync_copy(deg_v.at[pl.ds(half, half)], deg_sh.at[idxb], add=True)
    plsc.subcore_barrier()

    pltpu.sync_copy(acc_sh.at[stripe], out_h.at[c, stripe])

    @pl.when(c == 0)
    def _():
        pltpu.sync_copy(deg_sh.at[pl.ds(s * DSTRIPE, DSTRIPE)],
                        degout_h.at[pl.ds(s * DSTRIPE, DSTRIPE)])


_OUT_TYPES = (jax.ShapeDtypeStruct((NC, NPAD, DH), jnp.float32),
              jax.ShapeDtypeStruct((DROWS, DH), jnp.float32))

_sc_pass1 = functools.partial(
    pl.kernel, out_type=_OUT_TYPES, mesh=_mesh, compiler_params=_params,
    scratch_types=_scratch(False),
)(functools.partial(_sc_pass_body, False))

_sc_pass2 = functools.partial(
    pl.kernel, out_type=_OUT_TYPES, mesh=_mesh, compiler_params=_params,
    scratch_types=_scratch(True),
)(functools.partial(_sc_pass_body, True))


_ROWS_BLK = 400
_N_BLKS = N_NODES // _ROWS_BLK


def _dot_wt(a, w):
    return lax.dot_general(a, w, (((1,), (1,)), ((), ())),
                           preferred_element_type=jnp.float32)


def _final01_body(x_ref, y1_ref, dg_ref, w_ref, b_ref, o_ref):
    y1 = jnp.concatenate([y1_ref[0], y1_ref[1]], axis=1)
    o_ref[:, 0:D] = _dot_wt(x_ref[...], w_ref[0]) + b_ref[0, 0]
    o_ref[:, D:2 * D] = (_dot_wt(y1, w_ref[1])
                         + dg_ref[...] * b_ref[0, 1])


def _tc_final01(x, y1h, deg, w_all, b_all):
    return pl.pallas_call(
        _final01_body,
        grid=(_N_BLKS,),
        in_specs=[
            pl.BlockSpec((_ROWS_BLK, D), lambda i: (i, 0)),
            pl.BlockSpec((NC, _ROWS_BLK, DH), lambda i: (0, i, 0)),
            pl.BlockSpec((_ROWS_BLK, 1), lambda i: (i, 0)),
            pl.BlockSpec((3, D, D), lambda i: (0, 0, 0)),
            pl.BlockSpec((1, 3, D), lambda i: (0, 0, 0)),
        ],
        out_specs=pl.BlockSpec((_ROWS_BLK, 3 * D), lambda i: (i, 0)),
        out_shape=jax.ShapeDtypeStruct((N_NODES, 3 * D), jnp.float32),
    )(x, y1h, deg, w_all, b_all)


def _final2_body(y2_ref, dg2_ref, w_ref, b_ref, out01_ref, o_ref):
    del out01_ref
    y2 = jnp.concatenate([y2_ref[0], y2_ref[1]], axis=1)
    o_ref[...] = _dot_wt(y2, w_ref[2]) + dg2_ref[...] * b_ref[0, 2]


def _tc_final2(y2h, deg2, w_all, b_all, out01):
    return pl.pallas_call(
        _final2_body,
        grid=(_N_BLKS,),
        in_specs=[
            pl.BlockSpec((NC, _ROWS_BLK, DH), lambda i: (0, i, 0)),
            pl.BlockSpec((_ROWS_BLK, 1), lambda i: (i, 0)),
            pl.BlockSpec((3, D, D), lambda i: (0, 0, 0)),
            pl.BlockSpec((1, 3, D), lambda i: (0, 0, 0)),
            pl.BlockSpec(memory_space=pl.ANY),
        ],
        out_specs=pl.BlockSpec((_ROWS_BLK, D), lambda i: (i, 2)),
        out_shape=jax.ShapeDtypeStruct((N_NODES, 3 * D), jnp.float32),
        input_output_aliases={4: 0},
    )(y2h, deg2, w_all, b_all, out01)


def kernel(x, edge_index, W0, b0, W1, b1, W2, b2):
    x = x.astype(jnp.float32)

    src = edge_index[0].astype(jnp.int32)
    dst = edge_index[1].astype(jnp.int32)
    npad_e = EPAD - N_EDGES
    pad_dst = N_NODES + jnp.arange(npad_e, dtype=jnp.int32) % (NPAD - N_NODES)
    src2d = jnp.concatenate([src, jnp.zeros((npad_e,), jnp.int32)]
                            ).reshape(NS * CPT, CHUNK)
    dst2d = jnp.concatenate([dst, pad_dst]).reshape(NS * CPT, CHUNK)

    zeros = jnp.zeros((ROWS_PER_TILE, DH), jnp.float32)
    dummy_deg = jnp.zeros((DROWS, DH), jnp.float32)

    w_all = jnp.stack([W0, W1, W2]).astype(jnp.float32)
    b_all = jnp.stack([b0, b1, b2]).astype(jnp.float32)[None]

    y1h, deg = _sc_pass1(x, src2d, dst2d, zeros, dummy_deg)
    y2h, deg2 = _sc_pass2(y1h, src2d, dst2d, zeros, deg)
    out01 = _tc_final01(x, y1h, deg.reshape(NPAD, 1), w_all, b_all)
    return _tc_final2(y2h, deg2.reshape(NPAD, 1), w_all, b_all, out01)
